# baseline (speedup 1.0000x reference)
import jax
import jax.numpy as jnp
from jax import lax
from jax.experimental import pallas as pl
from jax.experimental.pallas import tpu as pltpu

_MIN_Y = 0.2
_MAX_Y = 0.8
_MIN_MM = 0.1
_MAX_MM = 0.2
_PAD_VALUE = -80.0
_MAXY = _MAX_Y - _MAX_MM


def _mask_params(b, h):
    key = jax.random.key(42)
    k1, k2, k3 = jax.random.split(key, 3)
    coin = jax.random.uniform(k1, (b,), dtype=jnp.float32)
    start_f = jax.random.uniform(k2, (b,), dtype=jnp.float32, minval=_MIN_Y, maxval=_MAXY)
    width_f = jax.random.uniform(k3, (b,), dtype=jnp.float32, minval=_MIN_MM, maxval=_MAX_MM)
    start = jnp.floor(start_f * h).astype(jnp.int32)
    width = jnp.floor(width_f * h).astype(jnp.int32)
    width = jnp.where(coin <= 1.0, width, 0)
    return start, width


def _body(start_ref, width_ref, x_ref, o_ref):
    i = pl.program_id(0)
    s = start_ref[i]
    e = s + width_ref[i]
    rows = lax.broadcasted_iota(jnp.int32, (1, 256, 2048), 1)
    o_ref[...] = jnp.where((rows >= s) & (rows < e), jnp.float32(_PAD_VALUE), x_ref[...])


def kernel(x):
    b, c, h, w = x.shape
    start, width = _mask_params(b, h)
    x3 = x.reshape(b, h, w)
    grid_spec = pltpu.PrefetchScalarGridSpec(
        num_scalar_prefetch=2,
        grid=(b,),
        in_specs=[pl.BlockSpec((1, h, w), lambda i, s_ref, w_ref: (i, 0, 0))],
        out_specs=pl.BlockSpec((1, h, w), lambda i, s_ref, w_ref: (i, 0, 0)),
    )
    out = pl.pallas_call(
        _body,
        grid_spec=grid_spec,
        out_shape=jax.ShapeDtypeStruct((b, h, w), jnp.float32),
    )(start, width, x3)
    return out.reshape(b, c, h, w)
